# baseline (speedup 1.0000x reference)
import jax
import jax.numpy as jnp
from jax.experimental import pallas as pl

_B, _N = 16, 2048
_D_EMB, _D_VIS, _D_BBOX, _D_KPT = 3072, 6, 4, 51
_FEAT = _D_EMB + _D_VIS + _D_BBOX + _D_KPT
_TOKEN_DIM = 128
_ROWS = 512
_CHUNKS = _N // _ROWS

_NT = (((1,), (1,)), ((), ()))


def _proj_kernel(emb_ref, vis_ref, bbox_ref, kpt_ref, mask_ref, w_ref, b_ref,
                 out_ref):
    w = w_ref[...]
    acc = jax.lax.dot_general(emb_ref[...], w[:, :_D_EMB], _NT,
                              preferred_element_type=jnp.float32)
    acc += jax.lax.dot_general(vis_ref[...], w[:, _D_EMB:_D_EMB + _D_VIS],
                               _NT, preferred_element_type=jnp.float32)
    acc += jax.lax.dot_general(
        bbox_ref[...], w[:, _D_EMB + _D_VIS:_D_EMB + _D_VIS + _D_BBOX],
        _NT, preferred_element_type=jnp.float32)
    acc += jax.lax.dot_general(kpt_ref[...], w[:, _D_EMB + _D_VIS + _D_BBOX:],
                               _NT, preferred_element_type=jnp.float32)
    acc += b_ref[...]
    i = pl.program_id(0)
    m = mask_ref[pl.ds(i // _CHUNKS, 1), pl.ds((i % _CHUNKS) * _ROWS, _ROWS)]
    out_ref[...] = acc * jnp.transpose(m, (1, 0))


def kernel(embeddings, visibility_scores, bbox_ltwh, keypoints_xyc,
           feats_masks, W, b):
    R = _B * _N
    emb = embeddings.reshape(R, _D_EMB)
    vis = visibility_scores.reshape(R, _D_VIS)
    bbox = bbox_ltwh.reshape(R, _D_BBOX)
    kpt = keypoints_xyc.reshape(R, _D_KPT)
    mask = feats_masks.astype(jnp.float32)
    b2 = b.reshape(1, _TOKEN_DIM)

    grid = (R // _ROWS,)
    out = pl.pallas_call(
        _proj_kernel,
        grid=grid,
        in_specs=[
            pl.BlockSpec((_ROWS, _D_EMB), lambda i: (i, 0)),
            pl.BlockSpec((_ROWS, _D_VIS), lambda i: (i, 0)),
            pl.BlockSpec((_ROWS, _D_BBOX), lambda i: (i, 0)),
            pl.BlockSpec((_ROWS, _D_KPT), lambda i: (i, 0)),
            pl.BlockSpec((_B, _N), lambda i: (0, 0)),
            pl.BlockSpec((_TOKEN_DIM, _FEAT), lambda i: (0, 0)),
            pl.BlockSpec((1, _TOKEN_DIM), lambda i: (0, 0)),
        ],
        out_specs=pl.BlockSpec((_ROWS, _TOKEN_DIM), lambda i: (i, 0)),
        out_shape=jax.ShapeDtypeStruct((R, _TOKEN_DIM), jnp.float32),
    )(emb, vis, bbox, kpt, mask, W, b2)

    return out.reshape(_B, _N, _TOKEN_DIM)


# speedup vs baseline: 1.2286x; 1.2286x over previous
import jax
import jax.numpy as jnp
from jax.experimental import pallas as pl

_B, _N = 16, 2048
_D_EMB, _D_VIS, _D_BBOX, _D_KPT = 3072, 6, 4, 51
_D_SMALL = _D_VIS + _D_BBOX + _D_KPT
_FEAT = _D_EMB + _D_SMALL
_TOKEN_DIM = 128
_ROWS = 512
_CHUNKS = _N // _ROWS


def _proj_kernel(emb_ref, vis_ref, bbox_ref, kpt_ref, mask_ref, wt_ref,
                 b_ref, out_ref):
    wt = wt_ref[...]
    acc = jnp.dot(emb_ref[...], wt[:_D_EMB],
                  preferred_element_type=jnp.float32)
    small = jnp.concatenate([vis_ref[...], bbox_ref[0], kpt_ref[...]], axis=0)
    acc += jnp.dot(jnp.transpose(small, (1, 0)), wt[_D_EMB:],
                   preferred_element_type=jnp.float32)
    acc += b_ref[...]
    i = pl.program_id(0)
    m = mask_ref[pl.ds(i // _CHUNKS, 1), pl.ds((i % _CHUNKS) * _ROWS, _ROWS)]
    out_ref[...] = acc * jnp.transpose(m, (1, 0))


def kernel(embeddings, visibility_scores, bbox_ltwh, keypoints_xyc,
           feats_masks, W, b):
    R = _B * _N
    emb = embeddings.reshape(R, _D_EMB)
    vis = jnp.transpose(visibility_scores, (2, 0, 1)).reshape(_D_VIS, R)
    bbox = jnp.transpose(bbox_ltwh, (0, 2, 1))
    kpt = jnp.transpose(keypoints_xyc, (2, 3, 0, 1)).reshape(_D_KPT, R)
    mask = feats_masks.astype(jnp.float32)
    wt = W.T
    b2 = b.reshape(1, _TOKEN_DIM)

    grid = (R // _ROWS,)
    out = pl.pallas_call(
        _proj_kernel,
        grid=grid,
        in_specs=[
            pl.BlockSpec((_ROWS, _D_EMB), lambda i: (i, 0)),
            pl.BlockSpec((_D_VIS, _ROWS), lambda i: (0, i)),
            pl.BlockSpec((1, _D_BBOX, _ROWS), lambda i: (i // _CHUNKS, 0, i % _CHUNKS)),
            pl.BlockSpec((_D_KPT, _ROWS), lambda i: (0, i)),
            pl.BlockSpec((_B, _N), lambda i: (0, 0)),
            pl.BlockSpec((_FEAT, _TOKEN_DIM), lambda i: (0, 0)),
            pl.BlockSpec((1, _TOKEN_DIM), lambda i: (0, 0)),
        ],
        out_specs=pl.BlockSpec((_ROWS, _TOKEN_DIM), lambda i: (i, 0)),
        out_shape=jax.ShapeDtypeStruct((R, _TOKEN_DIM), jnp.float32),
    )(emb, vis, bbox, kpt, mask, wt, b2)

    return out.reshape(_B, _N, _TOKEN_DIM)


# speedup vs baseline: 1.4465x; 1.1773x over previous
import jax
import jax.numpy as jnp
from jax.experimental import pallas as pl

_B, _N = 16, 2048
_D_EMB, _D_VIS, _D_BBOX, _D_KPT = 3072, 6, 4, 51
_D_SMALL = _D_VIS + _D_BBOX + _D_KPT
_FEAT = _D_EMB + _D_SMALL
_TOKEN_DIM = 128
_ROWS = 512
_CHUNKS = _N // _ROWS


def _proj_kernel(emb_ref, vis_ref, bbox_ref, kpt_ref, mask_ref, wt_ref,
                 b_ref, out_ref):
    i = pl.program_id(0)
    bi = i // _CHUNKS
    off = (i % _CHUNKS) * _ROWS

    wt = wt_ref[...]
    acc = jnp.dot(emb_ref[...], wt[:_D_EMB],
                  preferred_element_type=jnp.float32)
    v = vis_ref[:, bi, pl.ds(off, _ROWS)]
    bb = bbox_ref[bi, :, pl.ds(off, _ROWS)]
    kp = kpt_ref[:, bi, pl.ds(off, _ROWS)]
    small = jnp.concatenate([v, bb, kp], axis=0)
    acc += jnp.dot(jnp.transpose(small, (1, 0)), wt[_D_EMB:],
                   preferred_element_type=jnp.float32)
    acc += b_ref[...]
    m = mask_ref[pl.ds(bi, 1), pl.ds(off, _ROWS)]
    out_ref[...] = acc * jnp.transpose(m, (1, 0))


def kernel(embeddings, visibility_scores, bbox_ltwh, keypoints_xyc,
           feats_masks, W, b):
    R = _B * _N
    emb = embeddings.reshape(R, _D_EMB)
    vis = jnp.transpose(visibility_scores, (2, 0, 1))
    bbox = jnp.transpose(bbox_ltwh, (0, 2, 1))
    kpt = jnp.transpose(keypoints_xyc, (2, 3, 0, 1)).reshape(_D_KPT, _B, _N)
    mask = feats_masks.astype(jnp.float32)
    wt = W.T
    b2 = b.reshape(1, _TOKEN_DIM)

    grid = (R // _ROWS,)
    out = pl.pallas_call(
        _proj_kernel,
        grid=grid,
        in_specs=[
            pl.BlockSpec((_ROWS, _D_EMB), lambda i: (i, 0)),
            pl.BlockSpec((_D_VIS, _B, _N), lambda i: (0, 0, 0)),
            pl.BlockSpec((_B, _D_BBOX, _N), lambda i: (0, 0, 0)),
            pl.BlockSpec((_D_KPT, _B, _N), lambda i: (0, 0, 0)),
            pl.BlockSpec((_B, _N), lambda i: (0, 0)),
            pl.BlockSpec((_FEAT, _TOKEN_DIM), lambda i: (0, 0)),
            pl.BlockSpec((1, _TOKEN_DIM), lambda i: (0, 0)),
        ],
        out_specs=pl.BlockSpec((_ROWS, _TOKEN_DIM), lambda i: (i, 0)),
        out_shape=jax.ShapeDtypeStruct((R, _TOKEN_DIM), jnp.float32),
    )(emb, vis, bbox, kpt, mask, wt, b2)

    return out.reshape(_B, _N, _TOKEN_DIM)


# speedup vs baseline: 1.5964x; 1.1036x over previous
import jax
import jax.numpy as jnp
from jax.experimental import pallas as pl

_B, _N = 16, 2048
_D_EMB, _D_VIS, _D_BBOX, _D_KPT = 3072, 6, 4, 51
_D_SMALL = _D_VIS + _D_BBOX + _D_KPT
_FEAT = _D_EMB + _D_SMALL
_TOKEN_DIM = 128
_ROWS = 1024
_CHUNKS = _N // _ROWS


def _proj_kernel(emb_ref, vis_ref, bbox_ref, kpt_ref, mask_ref, wt_ref,
                 b_ref, out_ref):
    i = pl.program_id(0)
    bi = i // _CHUNKS
    off = (i % _CHUNKS) * _ROWS

    wt = wt_ref[...]
    acc = jnp.dot(emb_ref[...], wt[:_D_EMB],
                  preferred_element_type=jnp.float32)
    v = vis_ref[:, bi, pl.ds(off, _ROWS)]
    bb = bbox_ref[bi, :, pl.ds(off, _ROWS)]
    kp = kpt_ref[:, bi, pl.ds(off, _ROWS)]
    small = jnp.concatenate([v, bb, kp], axis=0)
    acc += jnp.dot(jnp.transpose(small, (1, 0)), wt[_D_EMB:],
                   preferred_element_type=jnp.float32)
    acc += b_ref[...]
    m = mask_ref[pl.ds(bi, 1), pl.ds(off, _ROWS)]
    out_ref[...] = acc * jnp.transpose(m, (1, 0))


def kernel(embeddings, visibility_scores, bbox_ltwh, keypoints_xyc,
           feats_masks, W, b):
    R = _B * _N
    emb = embeddings.reshape(R, _D_EMB)
    vis = jnp.transpose(visibility_scores, (2, 0, 1))
    bbox = jnp.transpose(bbox_ltwh, (0, 2, 1))
    kpt = jnp.transpose(keypoints_xyc, (2, 3, 0, 1)).reshape(_D_KPT, _B, _N)
    mask = feats_masks.astype(jnp.float32)
    wt = W.T
    b2 = b.reshape(1, _TOKEN_DIM)

    grid = (R // _ROWS,)
    out = pl.pallas_call(
        _proj_kernel,
        grid=grid,
        in_specs=[
            pl.BlockSpec((_ROWS, _D_EMB), lambda i: (i, 0)),
            pl.BlockSpec((_D_VIS, _B, _N), lambda i: (0, 0, 0)),
            pl.BlockSpec((_B, _D_BBOX, _N), lambda i: (0, 0, 0)),
            pl.BlockSpec((_D_KPT, _B, _N), lambda i: (0, 0, 0)),
            pl.BlockSpec((_B, _N), lambda i: (0, 0)),
            pl.BlockSpec((_FEAT, _TOKEN_DIM), lambda i: (0, 0)),
            pl.BlockSpec((1, _TOKEN_DIM), lambda i: (0, 0)),
        ],
        out_specs=pl.BlockSpec((_ROWS, _TOKEN_DIM), lambda i: (i, 0)),
        out_shape=jax.ShapeDtypeStruct((R, _TOKEN_DIM), jnp.float32),
    )(emb, vis, bbox, kpt, mask, wt, b2)

    return out.reshape(_B, _N, _TOKEN_DIM)


# speedup vs baseline: 1.5979x; 1.0010x over previous
import jax
import jax.numpy as jnp
from jax.experimental import pallas as pl

_B, _N = 16, 2048
_D_EMB, _D_VIS, _D_BBOX, _D_KPT = 3072, 6, 4, 51
_D_SMALL = _D_VIS + _D_BBOX + _D_KPT
_FEAT = _D_EMB + _D_SMALL
_TOKEN_DIM = 128
_ROWS = 1024
_CHUNKS = _N // _ROWS


def _proj_kernel(emb_ref, vis_ref, bbox_ref, kpt_ref, mask_ref, wt_ref,
                 b_ref, out_ref):
    i = pl.program_id(0)
    bi = i // _CHUNKS
    off = (i % _CHUNKS) * _ROWS

    wt = wt_ref[...]
    acc = jnp.dot(emb_ref[...], wt[:_D_EMB],
                  preferred_element_type=jnp.float32)
    v = vis_ref[:, bi, pl.ds(off, _ROWS)]
    bb = bbox_ref[bi, :, pl.ds(off, _ROWS)]
    kp = kpt_ref[:, bi, pl.ds(off, _ROWS)]
    small = jnp.concatenate([v, bb, kp], axis=0)
    acc += jax.lax.dot_general(small, wt[_D_EMB:], (((0,), (0,)), ((), ())),
                               preferred_element_type=jnp.float32)
    acc += b_ref[...]
    m = mask_ref[pl.ds(bi, 1), pl.ds(off, _ROWS)]
    out_ref[...] = acc * jnp.transpose(m, (1, 0))


def kernel(embeddings, visibility_scores, bbox_ltwh, keypoints_xyc,
           feats_masks, W, b):
    R = _B * _N
    emb = embeddings.reshape(R, _D_EMB)
    vis = jnp.transpose(visibility_scores, (2, 0, 1))
    bbox = jnp.transpose(bbox_ltwh, (0, 2, 1))
    kpt = jnp.transpose(keypoints_xyc, (2, 3, 0, 1)).reshape(_D_KPT, _B, _N)
    mask = feats_masks.astype(jnp.float32)
    wt = W.T
    b2 = b.reshape(1, _TOKEN_DIM)

    grid = (R // _ROWS,)
    out = pl.pallas_call(
        _proj_kernel,
        grid=grid,
        in_specs=[
            pl.BlockSpec((_ROWS, _D_EMB), lambda i: (i, 0)),
            pl.BlockSpec((_D_VIS, _B, _N), lambda i: (0, 0, 0)),
            pl.BlockSpec((_B, _D_BBOX, _N), lambda i: (0, 0, 0)),
            pl.BlockSpec((_D_KPT, _B, _N), lambda i: (0, 0, 0)),
            pl.BlockSpec((_B, _N), lambda i: (0, 0)),
            pl.BlockSpec((_FEAT, _TOKEN_DIM), lambda i: (0, 0)),
            pl.BlockSpec((1, _TOKEN_DIM), lambda i: (0, 0)),
        ],
        out_specs=pl.BlockSpec((_ROWS, _TOKEN_DIM), lambda i: (i, 0)),
        out_shape=jax.ShapeDtypeStruct((R, _TOKEN_DIM), jnp.float32),
    )(emb, vis, bbox, kpt, mask, wt, b2)

    return out.reshape(_B, _N, _TOKEN_DIM)
